# baseline (speedup 1.0000x reference)
import dataclasses
import functools

import jax
import jax.numpy as jnp
from jax import lax
from jax.experimental import pallas as pl
from jax.experimental.pallas import tpu as pltpu
from jax.experimental.pallas import tpu_sc as plsc

N = 10000
E = 320000
D = 128
EPS = 1e-6
NEG = 0.01

NC = 2
NS = 16
NW = NC * NS
EPW = E // NW
C = 80
NCH = EPW // C
DEPTH = 4
DDEPTH = 2 * DEPTH
SKEW = 2
NMAIN = 120
STRIPE = 632
LAST_STRIPE = N - STRIPE * (NS - 1)


def _leaky(a):
    return jnp.maximum(a, NEG * a)


def _mlp_body(x_ref, w1t_ref, w2t_ref, wot_ref, h_ref, t_ref):
    a = jnp.dot(x_ref[...], w1t_ref[...], preferred_element_type=jnp.float32)
    h = jnp.dot(_leaky(a), w2t_ref[...], preferred_element_type=jnp.float32)
    h_ref[...] = h
    t_ref[...] = jnp.dot(h, wot_ref[...], preferred_element_type=jnp.float32)


def _mlp(x, w1t, w2t, wot):
    return pl.pallas_call(
        _mlp_body,
        out_shape=(jax.ShapeDtypeStruct((N, D), jnp.float32),
                   jax.ShapeDtypeStruct((N, D), jnp.float32)),
    )(x, w1t, w2t, wot)


def _sc_agg_body(h_hbm, ei_hbm, w_hbm, z_hbm, out0_hbm, out1_hbm,
                 acc, ibs, ibd, ibw, rows0, rows1, rows2, rows3,
                 sem_is, sem_id, sem_iw, sem_g, sem_s):
    cid = lax.axis_index("c")
    sid = lax.axis_index("s")
    wid = sid * NC + cid
    rows = (rows0, rows1, rows2, rows3)

    off = sid * STRIPE

    @pl.when(sid < NS - 1)
    def _():
        s = pl.ds(off, STRIPE)
        pltpu.sync_copy(z_hbm.at[s], acc.at[s])

    @pl.when(sid == NS - 1)
    def _():
        s = pl.ds(off, LAST_STRIPE)
        pltpu.sync_copy(z_hbm.at[s], acc.at[s])

    plsc.subcore_barrier()

    base0 = wid * EPW

    def csl(g):
        return pl.ds(base0 + g * C, C)

    def ssl(g):
        return pl.ds(E + base0 + g * C, C)

    def start_idx(g, b):
        pltpu.async_copy(ei_hbm.at[ssl(g)], ibs.at[b], sem_is.at[b])
        pltpu.async_copy(ei_hbm.at[csl(g)], ibd.at[g & 7], sem_id.at[b])
        pltpu.async_copy(w_hbm.at[csl(g)], ibw.at[b], sem_iw.at[b])

    def a_stage(g, b, first=False):
        if not first:
            pltpu.make_async_copy(rows[b], acc.at[ibd.at[(g - DEPTH) & 7]],
                                  sem_s.at[b]).wait()
        pltpu.make_async_copy(ei_hbm.at[ssl(g)], ibs.at[b],
                              sem_is.at[b]).wait()
        pltpu.async_copy(h_hbm.at[ibs.at[b]], rows[b], sem_g.at[b])

    def b_stage(g, b, prefetch=True):
        rb = rows[b]
        pltpu.make_async_copy(h_hbm.at[ibs.at[b]], rb, sem_g.at[b]).wait()
        pltpu.make_async_copy(w_hbm.at[csl(g)], ibw.at[b],
                              sem_iw.at[b]).wait()
        bsel = jnp.full((16,), b, jnp.int32)

        @plsc.parallel_loop(0, C, unroll=4)
        def _edge(e):
            ws = plsc.load_gather(ibw, [bsel, jnp.full((16,), e, jnp.int32)])
            for k in range(D // 16):
                sl = (e, pl.ds(k * 16, 16))
                rb[sl] = rb[sl] * ws

        pltpu.make_async_copy(ei_hbm.at[csl(g)], ibd.at[g & 7],
                              sem_id.at[b]).wait()
        pltpu.async_copy(rb, acc.at[ibd.at[g & 7]], sem_s.at[b], add=True)
        if prefetch:
            start_idx(g + DEPTH, b)

    for g in range(DEPTH):
        start_idx(g, g)
    a_stage(0, 0, first=True)
    a_stage(1, 1, first=True)
    a_stage(2, 2, first=True)
    b_stage(0, 0)
    a_stage(3, 3, first=True)
    b_stage(1, 1)
    a_stage(4, 0)
    b_stage(2, 2)
    a_stage(5, 1)
    b_stage(3, 3)

    @pl.loop(1, NMAIN // DEPTH)
    def _block(j):
        t0 = DEPTH * j
        for b in range(DEPTH):
            a_stage(t0 + b + SKEW, (b + SKEW) % DEPTH)
            b_stage(t0 + b, b)

    a_stage(122, 2)
    b_stage(120, 0)
    a_stage(123, 3)
    b_stage(121, 1, prefetch=False)
    a_stage(124, 0)
    b_stage(122, 2, prefetch=False)
    b_stage(123, 3, prefetch=False)
    b_stage(124, 0, prefetch=False)
    for g in range(121, 125):
        b = g % DEPTH
        pltpu.make_async_copy(rows[b], acc.at[ibd.at[g & 7]],
                              sem_s.at[b]).wait()

    plsc.subcore_barrier()
    plsc.subcore_barrier()

    sz = sid * STRIPE
    for c, out_hbm in ((0, out0_hbm), (1, out1_hbm)):
        @pl.when((cid == c) & (sid < NS - 1))
        def _():
            s = pl.ds(sz, STRIPE)
            pltpu.sync_copy(acc.at[s], out_hbm.at[s])

        @pl.when((cid == c) & (sid == NS - 1))
        def _():
            s = pl.ds(sz, LAST_STRIPE)
            pltpu.sync_copy(acc.at[s], out_hbm.at[s])


@functools.cache
def _sc_agg_kernel():
    cp = pltpu.CompilerParams()
    if "needs_layout_passes" in pltpu.CompilerParams.__dataclass_fields__:
        cp = dataclasses.replace(cp, needs_layout_passes=False)
    return pl.kernel(
        _sc_agg_body,
        compiler_params=cp,
        mesh=plsc.VectorSubcoreMesh(core_axis_name="c", subcore_axis_name="s"),
        out_type=(jax.ShapeDtypeStruct((N, D), jnp.float32),
                  jax.ShapeDtypeStruct((N, D), jnp.float32)),
        scratch_types=[
            pltpu.VMEM_SHARED((N, D), jnp.float32),
            pltpu.VMEM((DEPTH, C), jnp.int32),
            pltpu.VMEM((DDEPTH, C), jnp.int32),
            pltpu.VMEM((DEPTH, C), jnp.float32),
            pltpu.VMEM((C, D), jnp.float32),
            pltpu.VMEM((C, D), jnp.float32),
            pltpu.VMEM((C, D), jnp.float32),
            pltpu.VMEM((C, D), jnp.float32),
            pltpu.SemaphoreType.DMA((DEPTH,)),
            pltpu.SemaphoreType.DMA((DEPTH,)),
            pltpu.SemaphoreType.DMA((DEPTH,)),
            pltpu.SemaphoreType.DMA((DEPTH,)),
            pltpu.SemaphoreType.DMA((DEPTH,)),
        ],
    )


def _finish_body(t_ref, p0_ref, p1_ref, wnt_ref, g_ref, b_ref, o_ref):
    xn = _leaky(p0_ref[...] + p1_ref[...])
    y = t_ref[...] + jnp.dot(xn, wnt_ref[...],
                             preferred_element_type=jnp.float32)
    mu = jnp.sum(y, axis=0, keepdims=True) * (1.0 / N)
    d = y - mu
    var = jnp.sum(d * d, axis=0, keepdims=True) * (1.0 / (N - 1))
    gam = jnp.reshape(g_ref[...], (1, D))
    bet = jnp.reshape(b_ref[...], (1, D))
    o_ref[...] = d / (jnp.sqrt(var) + EPS) * gam + bet


def _finish(t, p0, p1, wnt, gamma, beta):
    return pl.pallas_call(
        _finish_body,
        out_shape=jax.ShapeDtypeStruct((N, D), jnp.float32),
    )(t, p0, p1, wnt, gamma, beta)


def kernel(x, edge_index, w, batch, batch_num, W1, W2, Wo, Wn, gamma, beta):
    eflat = jnp.reshape(edge_index, (2 * E,))
    wf = jnp.reshape(w, (E,))
    h, t = _mlp(x, W1.T, W2.T, Wo.T)
    z = jnp.zeros((N, D), jnp.float32)
    p0, p1 = _sc_agg_kernel()(h, eflat, wf, z)
    return _finish(t, p0, p1, Wn.T, gamma, beta)


# speedup vs baseline: 1.0197x; 1.0197x over previous
import dataclasses
import functools

import jax
import jax.numpy as jnp
from jax import lax
from jax.experimental import pallas as pl
from jax.experimental.pallas import tpu as pltpu
from jax.experimental.pallas import tpu_sc as plsc

N = 10000
E = 320000
D = 128
EPS = 1e-6
NEG = 0.01

NC = 2
NS = 16
NW = NC * NS
EPW = E // NW
C = 80
NCH = EPW // C
DEPTH = 4
DDEPTH = 2 * DEPTH
SKEW = 2
NMAIN = 120
STRIPE = 632
LAST_STRIPE = N - STRIPE * (NS - 1)


def _leaky(a):
    return jnp.maximum(a, NEG * a)


def _mlp_body(x_ref, w1t_ref, w2t_ref, wot_ref, h_ref, t_ref):
    a = jnp.dot(x_ref[...], w1t_ref[...], preferred_element_type=jnp.float32)
    h = jnp.dot(_leaky(a), w2t_ref[...], preferred_element_type=jnp.float32)
    h_ref[...] = h
    t_ref[...] = jnp.dot(h, wot_ref[...], preferred_element_type=jnp.float32)


def _mlp(x, w1t, w2t, wot):
    return pl.pallas_call(
        _mlp_body,
        out_shape=(jax.ShapeDtypeStruct((N, D), jnp.float32),
                   jax.ShapeDtypeStruct((N, D), jnp.float32)),
    )(x, w1t, w2t, wot)


def _sc_agg_body(h_hbm, ei_hbm, w_hbm, z_hbm, out0_hbm, out1_hbm,
                 acc, ibs, ibd, ibw, rows0, rows1, rows2, rows3,
                 sem_is, sem_id, sem_iw, sem_g, sem_s, sem_z):
    cid = lax.axis_index("c")
    sid = lax.axis_index("s")
    wid = sid * NC + cid
    rows = (rows0, rows1, rows2, rows3)

    off = sid * STRIPE

    @pl.when(sid < NS - 1)
    def _():
        s = pl.ds(off, STRIPE)
        pltpu.async_copy(z_hbm.at[s], acc.at[s], sem_z)

    @pl.when(sid == NS - 1)
    def _():
        s = pl.ds(off, LAST_STRIPE)
        pltpu.async_copy(z_hbm.at[s], acc.at[s], sem_z)

    base0 = wid * EPW

    def csl(g):
        return pl.ds(base0 + g * C, C)

    def ssl(g):
        return pl.ds(E + base0 + g * C, C)

    def start_idx(g, b):
        pltpu.async_copy(ei_hbm.at[ssl(g)], ibs.at[b], sem_is.at[b])
        pltpu.async_copy(ei_hbm.at[csl(g)], ibd.at[g & 7], sem_id.at[b])
        pltpu.async_copy(w_hbm.at[csl(g)], ibw.at[b], sem_iw.at[b])

    def a_stage(g, b, first=False):
        if not first:
            pltpu.make_async_copy(rows[b], acc.at[ibd.at[(g - DEPTH) & 7]],
                                  sem_s.at[b]).wait()
        pltpu.make_async_copy(ei_hbm.at[ssl(g)], ibs.at[b],
                              sem_is.at[b]).wait()
        pltpu.async_copy(h_hbm.at[ibs.at[b]], rows[b], sem_g.at[b])

    def b_stage(g, b, prefetch=True):
        rb = rows[b]
        pltpu.make_async_copy(h_hbm.at[ibs.at[b]], rb, sem_g.at[b]).wait()
        pltpu.make_async_copy(w_hbm.at[csl(g)], ibw.at[b],
                              sem_iw.at[b]).wait()
        bsel = jnp.full((16,), b, jnp.int32)

        @plsc.parallel_loop(0, C, unroll=2)
        def _edge(e):
            ws = plsc.load_gather(ibw, [bsel, jnp.full((16,), e, jnp.int32)])
            for k in range(D // 16):
                sl = (e, pl.ds(k * 16, 16))
                rb[sl] = rb[sl] * ws

        pltpu.make_async_copy(ei_hbm.at[csl(g)], ibd.at[g & 7],
                              sem_id.at[b]).wait()
        pltpu.async_copy(rb, acc.at[ibd.at[g & 7]], sem_s.at[b], add=True)
        if prefetch:
            start_idx(g + DEPTH, b)

    for g in range(DEPTH):
        start_idx(g, g)
    a_stage(0, 0, first=True)
    a_stage(1, 1, first=True)
    a_stage(2, 2, first=True)

    @pl.when(sid < NS - 1)
    def _():
        s = pl.ds(off, STRIPE)
        pltpu.make_async_copy(z_hbm.at[s], acc.at[s], sem_z).wait()

    @pl.when(sid == NS - 1)
    def _():
        s = pl.ds(off, LAST_STRIPE)
        pltpu.make_async_copy(z_hbm.at[s], acc.at[s], sem_z).wait()

    plsc.subcore_barrier()
    b_stage(0, 0)
    a_stage(3, 3, first=True)
    b_stage(1, 1)
    a_stage(4, 0)
    b_stage(2, 2)
    a_stage(5, 1)
    b_stage(3, 3)

    @pl.loop(1, NMAIN // DEPTH)
    def _block(j):
        t0 = DEPTH * j
        for b in range(DEPTH):
            a_stage(t0 + b + SKEW, (b + SKEW) % DEPTH)
            b_stage(t0 + b, b)

    a_stage(122, 2)
    b_stage(120, 0)
    a_stage(123, 3)
    b_stage(121, 1, prefetch=False)
    a_stage(124, 0)
    b_stage(122, 2, prefetch=False)
    b_stage(123, 3, prefetch=False)
    b_stage(124, 0, prefetch=False)
    for g in range(121, 125):
        b = g % DEPTH
        pltpu.make_async_copy(rows[b], acc.at[ibd.at[g & 7]],
                              sem_s.at[b]).wait()

    plsc.subcore_barrier()
    plsc.subcore_barrier()

    sz = sid * STRIPE
    for c, out_hbm in ((0, out0_hbm), (1, out1_hbm)):
        @pl.when((cid == c) & (sid < NS - 1))
        def _():
            s = pl.ds(sz, STRIPE)
            pltpu.sync_copy(acc.at[s], out_hbm.at[s])

        @pl.when((cid == c) & (sid == NS - 1))
        def _():
            s = pl.ds(sz, LAST_STRIPE)
            pltpu.sync_copy(acc.at[s], out_hbm.at[s])


@functools.cache
def _sc_agg_kernel():
    cp = pltpu.CompilerParams()
    if "needs_layout_passes" in pltpu.CompilerParams.__dataclass_fields__:
        cp = dataclasses.replace(cp, needs_layout_passes=False)
    return pl.kernel(
        _sc_agg_body,
        compiler_params=cp,
        mesh=plsc.VectorSubcoreMesh(core_axis_name="c", subcore_axis_name="s"),
        out_type=(jax.ShapeDtypeStruct((N, D), jnp.float32),
                  jax.ShapeDtypeStruct((N, D), jnp.float32)),
        scratch_types=[
            pltpu.VMEM_SHARED((N, D), jnp.float32),
            pltpu.VMEM((DEPTH, C), jnp.int32),
            pltpu.VMEM((DDEPTH, C), jnp.int32),
            pltpu.VMEM((DEPTH, C), jnp.float32),
            pltpu.VMEM((C, D), jnp.float32),
            pltpu.VMEM((C, D), jnp.float32),
            pltpu.VMEM((C, D), jnp.float32),
            pltpu.VMEM((C, D), jnp.float32),
            pltpu.SemaphoreType.DMA((DEPTH,)),
            pltpu.SemaphoreType.DMA((DEPTH,)),
            pltpu.SemaphoreType.DMA((DEPTH,)),
            pltpu.SemaphoreType.DMA((DEPTH,)),
            pltpu.SemaphoreType.DMA((DEPTH,)),
            pltpu.SemaphoreType.DMA,
        ],
    )


def _finish_body(t_ref, p0_ref, p1_ref, wnt_ref, g_ref, b_ref, o_ref):
    xn = _leaky(p0_ref[...] + p1_ref[...])
    y = t_ref[...] + jnp.dot(xn, wnt_ref[...],
                             preferred_element_type=jnp.float32)
    mu = jnp.sum(y, axis=0, keepdims=True) * (1.0 / N)
    d = y - mu
    var = jnp.sum(d * d, axis=0, keepdims=True) * (1.0 / (N - 1))
    gam = jnp.reshape(g_ref[...], (1, D))
    bet = jnp.reshape(b_ref[...], (1, D))
    o_ref[...] = d / (jnp.sqrt(var) + EPS) * gam + bet


def _finish(t, p0, p1, wnt, gamma, beta):
    return pl.pallas_call(
        _finish_body,
        out_shape=jax.ShapeDtypeStruct((N, D), jnp.float32),
    )(t, p0, p1, wnt, gamma, beta)


def kernel(x, edge_index, w, batch, batch_num, W1, W2, Wo, Wn, gamma, beta):
    eflat = jnp.reshape(edge_index, (2 * E,))
    wf = jnp.reshape(w, (E,))
    h, t = _mlp(x, W1.T, W2.T, Wo.T)
    z = jnp.zeros((N, D), jnp.float32)
    p0, p1 = _sc_agg_kernel()(h, eflat, wf, z)
    return _finish(t, p0, p1, Wn.T, gamma, beta)
